# baseline (speedup 1.0000x reference)
import functools

import jax
import jax.numpy as jnp
from jax import lax
from jax.experimental import pallas as pl
from jax.experimental.pallas import tpu as pltpu
from jax.experimental.pallas import tpu_sc as plsc

_N = 10000
_E = 160000
_D_IN = 256
_D_H = 512
_D_OUT = 256

_NSUB = 16
_NCORE = 2
_NW = _NSUB * _NCORE
_NPAD = 10240
_ZROWS = _NPAD // _NSUB
_K = 128
_W_E = _E // _NW
_NCHUNK = -(-_W_E // _K)
_PAD_W = _NCHUNK * _K


def _make_agg(d, with_deg):
  mesh = plsc.VectorSubcoreMesh(core_axis_name="core", subcore_axis_name="subcore")
  out_type = [jax.ShapeDtypeStruct((_NW, _NPAD, d), jnp.float32)]
  if with_deg:
    out_type.append(jax.ShapeDtypeStruct((_NW, _NPAD, _D_IN), jnp.float32))
  scratch = [
      pltpu.VMEM((_K,), jnp.int32),
      pltpu.VMEM((_K,), jnp.int32),
      pltpu.VMEM((_K, d), jnp.float32),
  ]
  if with_deg:
    scratch.append(pltpu.VMEM((_K, _D_IN), jnp.float32))

  def body(*refs):
    if with_deg:
      (table, src_h, dst_h, zrow, ones_h, part_o, deg_o,
       src_v, dst_v, rows_v, ones_v) = refs
    else:
      (table, src_h, dst_h, zrow, part_o, src_v, dst_v, rows_v) = refs
    c = lax.axis_index("core")
    s = lax.axis_index("subcore")
    w = c * _NSUB + s

    @pl.loop(0, _NSUB)
    def _(zi):
      pltpu.sync_copy(zrow, part_o.at[w].at[pl.ds(zi * _ZROWS, _ZROWS)])
      if with_deg:
        pltpu.sync_copy(zrow, deg_o.at[w].at[pl.ds(zi * _ZROWS, _ZROWS)])
    if with_deg:
      pltpu.sync_copy(ones_h, ones_v)

    @pl.loop(0, _NCHUNK)
    def _(ci):
      eoff = w * _PAD_W + ci * _K
      pltpu.sync_copy(src_h.at[pl.ds(eoff, _K)], src_v)
      pltpu.sync_copy(dst_h.at[pl.ds(eoff, _K)], dst_v)
      pltpu.sync_copy(table.at[src_v], rows_v)
      pltpu.sync_copy(rows_v, part_o.at[w].at[dst_v], add=True)
      if with_deg:
        pltpu.sync_copy(ones_v, deg_o.at[w].at[dst_v], add=True)

  cp = pltpu.CompilerParams(needs_layout_passes=False)
  return pl.kernel(body, out_type=out_type, mesh=mesh, scratch_types=scratch,
                   compiler_params=cp)


def _tc1_body(part, deg, x, w1l, w1r, b1, w2l, h_o, p_o):
  agg = part[0] + part[1]
  d16 = deg[0] + deg[1]
  inv = 1.0 / jnp.maximum(d16[:, :1], 1.0)
  mean = agg * inv
  pre = (jnp.dot(mean, w1l[...], preferred_element_type=jnp.float32)
         + jnp.dot(x[...], w1r[...], preferred_element_type=jnp.float32)
         + b1[...])
  h = jnp.maximum(pre, 0.0)
  h_o[...] = h
  p_o[...] = jnp.dot(h, w2l[...], preferred_element_type=jnp.float32)


def _tc2_body(part, deg, h, w2r, b2, o):
  agg = part[0] + part[1]
  d16 = deg[0] + deg[1]
  inv = 1.0 / jnp.maximum(d16[:, :1], 1.0)
  pre = (agg * inv
         + jnp.dot(h[...], w2r[...], preferred_element_type=jnp.float32)
         + b2[...])
  m = jnp.max(pre, axis=1, keepdims=True)
  e = jnp.exp(pre - m)
  lse = jnp.log(jnp.sum(e, axis=1, keepdims=True))
  o[...] = pre - m - lse


_RB = 640


def _tc1(part, deg, x, w1l, w1r, b1, w2l):
  nb = _NPAD // _RB
  return pl.pallas_call(
      _tc1_body,
      grid=(nb,),
      in_specs=[
          pl.BlockSpec((_NCORE, _RB, _D_IN), lambda i: (0, i, 0)),
          pl.BlockSpec((_NCORE, _RB, _D_IN), lambda i: (0, i, 0)),
          pl.BlockSpec((_RB, _D_IN), lambda i: (i, 0)),
          pl.BlockSpec((_D_IN, _D_H), lambda i: (0, 0)),
          pl.BlockSpec((_D_IN, _D_H), lambda i: (0, 0)),
          pl.BlockSpec((1, _D_H), lambda i: (0, 0)),
          pl.BlockSpec((_D_H, _D_OUT), lambda i: (0, 0)),
      ],
      out_specs=[
          pl.BlockSpec((_RB, _D_H), lambda i: (i, 0)),
          pl.BlockSpec((_RB, _D_OUT), lambda i: (i, 0)),
      ],
      out_shape=[jax.ShapeDtypeStruct((_NPAD, _D_H), jnp.float32),
                 jax.ShapeDtypeStruct((_NPAD, _D_OUT), jnp.float32)],
  )(part, deg, x, w1l, w1r, b1, w2l)


def _tc2(part, deg, h, w2r, b2):
  nb = _NPAD // _RB
  return pl.pallas_call(
      _tc2_body,
      grid=(nb,),
      in_specs=[
          pl.BlockSpec((_NCORE, _RB, _D_OUT), lambda i: (0, i, 0)),
          pl.BlockSpec((_NCORE, _RB, _D_IN), lambda i: (0, i, 0)),
          pl.BlockSpec((_RB, _D_H), lambda i: (i, 0)),
          pl.BlockSpec((_D_H, _D_OUT), lambda i: (0, 0)),
          pl.BlockSpec((1, _D_OUT), lambda i: (0, 0)),
      ],
      out_specs=pl.BlockSpec((_RB, _D_OUT), lambda i: (i, 0)),
      out_shape=jax.ShapeDtypeStruct((_NPAD, _D_OUT), jnp.float32),
  )(part, deg, h, w2r, b2)


def kernel(x, edge_index, W1l, W1r, b1, W2l, W2r, b2):
  pad = _PAD_W - _W_E
  src = jnp.pad(edge_index[0].reshape(_NW, _W_E), ((0, 0), (0, pad)),
                constant_values=0).reshape(-1)
  dst = jnp.pad(edge_index[1].reshape(_NW, _W_E), ((0, 0), (0, pad)),
                constant_values=_N).reshape(-1)
  x_pad = jnp.pad(x, ((0, _NPAD - _N), (0, 0)))
  zrow = jnp.zeros((_ZROWS, _D_IN), jnp.float32)
  ones_k = jnp.ones((_K, _D_IN), jnp.float32)

  part1, degp = _make_agg(_D_IN, True)(x_pad, src, dst, zrow, ones_k)
  part1 = jnp.stack([part1[0::2].sum(0), part1[1::2].sum(0)])
  degp = jnp.stack([degp[0::2].sum(0), degp[1::2].sum(0)])
  h, p = _tc1(part1, degp, x_pad, W1l, W1r, b1.reshape(1, _D_H), W2l)
  (part2,) = _make_agg(_D_OUT, False)(p, src, dst, zrow)
  part2 = jnp.stack([part2[0::2].sum(0), part2[1::2].sum(0)])
  out = _tc2(part2, degp, h, W2r, b2.reshape(1, _D_OUT))
  return out[:_N]


# speedup vs baseline: 11.5236x; 11.5236x over previous
import functools

import jax
import jax.numpy as jnp
from jax import lax
from jax.experimental import pallas as pl
from jax.experimental.pallas import tpu as pltpu
from jax.experimental.pallas import tpu_sc as plsc

_N = 10000
_E = 160000
_D_IN = 256
_D_H = 512
_D_OUT = 256

_NSUB = 16
_NCORE = 2
_NW = _NSUB * _NCORE
_NPAD = 10240
_ZROWS = _NPAD // _NSUB
_K = 128
_W_E = _E // _NW
_NCHUNK = -(-_W_E // _K)
_PAD_W = _NCHUNK * _K


def _make_agg(d, with_deg):
  mesh = plsc.VectorSubcoreMesh(core_axis_name="core", subcore_axis_name="subcore")
  out_type = [jax.ShapeDtypeStruct((_NCORE, _NPAD, d), jnp.float32)]
  if with_deg:
    out_type.append(jax.ShapeDtypeStruct((_NCORE, _NPAD, _D_IN), jnp.float32))
  scratch = [
      pltpu.VMEM((_K,), jnp.int32),
      pltpu.VMEM((_K,), jnp.int32),
      pltpu.VMEM((_K, d), jnp.float32),
  ]
  if with_deg:
    scratch.append(pltpu.VMEM((_K, _D_IN), jnp.float32))

  def body(*refs):
    if with_deg:
      (table, src_h, dst_h, zrow, ones_h, part_o, deg_o,
       src_v, dst_v, rows_v, ones_v) = refs
    else:
      (table, src_h, dst_h, zrow, part_o, src_v, dst_v, rows_v) = refs
    c = lax.axis_index("core")
    s = lax.axis_index("subcore")
    w = c * _NSUB + s

    pltpu.sync_copy(zrow, part_o.at[c].at[pl.ds(s * _ZROWS, _ZROWS)])
    if with_deg:
      pltpu.sync_copy(zrow, deg_o.at[c].at[pl.ds(s * _ZROWS, _ZROWS)])
      pltpu.sync_copy(ones_h, ones_v)
    plsc.subcore_barrier()

    @pl.loop(0, _NCHUNK)
    def _(ci):
      eoff = w * _PAD_W + ci * _K
      pltpu.sync_copy(src_h.at[pl.ds(eoff, _K)], src_v)
      pltpu.sync_copy(dst_h.at[pl.ds(eoff, _K)], dst_v)
      pltpu.sync_copy(table.at[src_v], rows_v)
      pltpu.sync_copy(rows_v, part_o.at[c].at[dst_v], add=True)
      if with_deg:
        pltpu.sync_copy(ones_v, deg_o.at[c].at[dst_v], add=True)

  cp = pltpu.CompilerParams(needs_layout_passes=False)
  return pl.kernel(body, out_type=out_type, mesh=mesh, scratch_types=scratch,
                   compiler_params=cp)


def _tc1_body(part, deg, x, w1l, w1r, b1, w2l, h_o, p_o):
  agg = part[0] + part[1]
  d16 = deg[0] + deg[1]
  inv = 1.0 / jnp.maximum(d16[:, :1], 1.0)
  mean = agg * inv
  pre = (jnp.dot(mean, w1l[...], preferred_element_type=jnp.float32)
         + jnp.dot(x[...], w1r[...], preferred_element_type=jnp.float32)
         + b1[...])
  h = jnp.maximum(pre, 0.0)
  h_o[...] = h
  p_o[...] = jnp.dot(h, w2l[...], preferred_element_type=jnp.float32)


def _tc2_body(part, deg, h, w2r, b2, o):
  agg = part[0] + part[1]
  d16 = deg[0] + deg[1]
  inv = 1.0 / jnp.maximum(d16[:, :1], 1.0)
  pre = (agg * inv
         + jnp.dot(h[...], w2r[...], preferred_element_type=jnp.float32)
         + b2[...])
  m = jnp.max(pre, axis=1, keepdims=True)
  e = jnp.exp(pre - m)
  lse = jnp.log(jnp.sum(e, axis=1, keepdims=True))
  o[...] = pre - m - lse


_RB = 640


def _tc1(part, deg, x, w1l, w1r, b1, w2l):
  nb = _NPAD // _RB
  return pl.pallas_call(
      _tc1_body,
      grid=(nb,),
      in_specs=[
          pl.BlockSpec((_NCORE, _RB, _D_IN), lambda i: (0, i, 0)),
          pl.BlockSpec((_NCORE, _RB, _D_IN), lambda i: (0, i, 0)),
          pl.BlockSpec((_RB, _D_IN), lambda i: (i, 0)),
          pl.BlockSpec((_D_IN, _D_H), lambda i: (0, 0)),
          pl.BlockSpec((_D_IN, _D_H), lambda i: (0, 0)),
          pl.BlockSpec((1, _D_H), lambda i: (0, 0)),
          pl.BlockSpec((_D_H, _D_OUT), lambda i: (0, 0)),
      ],
      out_specs=[
          pl.BlockSpec((_RB, _D_H), lambda i: (i, 0)),
          pl.BlockSpec((_RB, _D_OUT), lambda i: (i, 0)),
      ],
      out_shape=[jax.ShapeDtypeStruct((_NPAD, _D_H), jnp.float32),
                 jax.ShapeDtypeStruct((_NPAD, _D_OUT), jnp.float32)],
  )(part, deg, x, w1l, w1r, b1, w2l)


def _tc2(part, deg, h, w2r, b2):
  nb = _NPAD // _RB
  return pl.pallas_call(
      _tc2_body,
      grid=(nb,),
      in_specs=[
          pl.BlockSpec((_NCORE, _RB, _D_OUT), lambda i: (0, i, 0)),
          pl.BlockSpec((_NCORE, _RB, _D_IN), lambda i: (0, i, 0)),
          pl.BlockSpec((_RB, _D_H), lambda i: (i, 0)),
          pl.BlockSpec((_D_H, _D_OUT), lambda i: (0, 0)),
          pl.BlockSpec((1, _D_OUT), lambda i: (0, 0)),
      ],
      out_specs=pl.BlockSpec((_RB, _D_OUT), lambda i: (i, 0)),
      out_shape=jax.ShapeDtypeStruct((_NPAD, _D_OUT), jnp.float32),
  )(part, deg, h, w2r, b2)


def kernel(x, edge_index, W1l, W1r, b1, W2l, W2r, b2):
  pad = _PAD_W - _W_E
  src = jnp.pad(edge_index[0].reshape(_NW, _W_E), ((0, 0), (0, pad)),
                constant_values=0).reshape(-1)
  dst = jnp.pad(edge_index[1].reshape(_NW, _W_E), ((0, 0), (0, pad)),
                constant_values=_N).reshape(-1)
  x_pad = jnp.pad(x, ((0, _NPAD - _N), (0, 0)))
  zrow = jnp.zeros((_ZROWS, _D_IN), jnp.float32)
  ones_k = jnp.ones((_K, _D_IN), jnp.float32)

  part1, degp = _make_agg(_D_IN, True)(x_pad, src, dst, zrow, ones_k)
  h, p = _tc1(part1, degp, x_pad, W1l, W1r, b1.reshape(1, _D_H), W2l)
  (part2,) = _make_agg(_D_OUT, False)(p, src, dst, zrow)
  out = _tc2(part2, degp, h, W2r, b2.reshape(1, _D_OUT))
  return out[:_N]
